# baseline (speedup 1.0000x reference)
import functools

import jax
import jax.numpy as jnp
from jax import lax
from jax.experimental import pallas as pl
from jax.experimental.pallas import tpu as pltpu
from jax.experimental.pallas import tpu_sc as plsc

N_USERS_C = 50000
N_ITEMS_C = 50000
N_TOTAL = N_USERS_C + N_ITEMS_C
DIM = 32
HALF = 16
NNZ_C = 1600000
BATCH_C = 4096
N_LAYERS_C = 3

NUM_TILES = 16
CHUNK = 128
SUBCH = 4
SUPER = CHUNK * SUBCH
SUPERS_PER_TILE = 201
EDGES_PER_TILE = SUPER * SUPERS_PER_TILE
NNZ_PAD = EDGES_PER_TILE * NUM_TILES
ROWS_PER_TILE = 6300
ACC_ROWS = ROWS_PER_TILE * NUM_TILES


def _make_spmm():
    mesh = plsc.VectorSubcoreMesh(core_axis_name="c", subcore_axis_name="s")

    @functools.partial(
        pl.kernel,
        mesh=mesh,
        compiler_params=pltpu.CompilerParams(
            needs_layout_passes=False, use_tc_tiling_on_sc=False),
        out_type=jax.ShapeDtypeStruct((2, ACC_ROWS, HALF), jnp.float32),
        scratch_types=(
            [pltpu.VMEM((2, SUPER), jnp.int32) for _ in range(3)]
            + [pltpu.VMEM((SUPER,), jnp.float32) for _ in range(3)]
            + [pltpu.VMEM((SUPER, HALF), jnp.float32) for _ in range(3)]
            + [pltpu.VMEM_SHARED((ACC_ROWS, HALF), jnp.float32)]
            + [pltpu.SemaphoreType.DMA for _ in range(9)]
        ),
    )
    def spmm(lo_hbm, hi_hbm, cr_hbm, val_hbm, zero_hbm, out_hbm,
             cr0, cr1, cr2, v0, v1, v2, g0, g1, g2, acc,
             ls0, ls1, ls2, gs0, gs1, gs2, ss0, ss1, ss2):
        cr_v = [cr0, cr1, cr2]
        val_v = [v0, v1, v2]
        rows_v = [g0, g1, g2]
        lsem = [ls0, ls1, ls2]
        gsem = [gs0, gs1, gs2]
        ssem = [ss0, ss1, ss2]
        c = lax.axis_index("c")
        s = lax.axis_index("s")
        rbase = s * ROWS_PER_TILE
        pltpu.sync_copy(zero_hbm, acc.at[pl.ds(rbase, ROWS_PER_TILE)])
        plsc.subcore_barrier()

        sbase = s * SUPERS_PER_TILE
        esplats = [jnp.full((16,), e, jnp.int32) for e in range(16)]

        def edge_loop(table):
            def fire_l(t, m):
                sc = sbase + m
                pltpu.async_copy(cr_hbm.at[sc], cr_v[t], lsem[t])
                pltpu.async_copy(val_hbm.at[pl.ds(sc * SUPER, SUPER)], val_v[t], lsem[t])

            def drain_l(t, m):
                sc = sbase + m
                pltpu.make_async_copy(cr_hbm.at[sc], cr_v[t], lsem[t]).wait()
                pltpu.make_async_copy(val_hbm.at[pl.ds(sc * SUPER, SUPER)], val_v[t], lsem[t]).wait()

            def fire_g(t):
                pltpu.async_copy(table.at[cr_v[t].at[0]], rows_v[t], gsem[t])

            def drain_g(t):
                pltpu.make_async_copy(table.at[cr_v[t].at[0]], rows_v[t], gsem[t]).wait()

            def fire_s(t):
                pltpu.async_copy(rows_v[t], acc.at[cr_v[t].at[1]], ssem[t], add=True)

            def drain_s(t):
                pltpu.make_async_copy(rows_v[t], acc.at[cr_v[t].at[1]], ssem[t]).wait()

            def scale(t):
                def body(g, _):
                    vv16 = val_v[t][pl.ds(g * 16, 16)]
                    for e in range(16):
                        r = g * 16 + e
                        vv = jnp.take(vv16, esplats[e])
                        rows_v[t][r, :] = rows_v[t][r, :] * vv
                    return 0

                lax.fori_loop(0, SUPER // 16, body, 0)

            def half(m, t, with_drain_s=True, with_next=True, with_next2=True):
                t1 = (t + 1) % 3
                t2 = (t + 2) % 3
                if with_next:
                    drain_l(t1, m + 1)
                    fire_g(t1)
                drain_g(t)
                scale(t)
                fire_s(t)
                if with_drain_s:
                    drain_s(t2)
                if with_next2:
                    fire_l(t2, m + 2)

            fire_l(0, 0)
            drain_l(0, 0)
            fire_g(0)
            fire_l(1, 1)
            half(0, 0, with_drain_s=False)

            def triple(k, _):
                m = 3 * k + 1
                half(m, 1)
                half(m + 1, 2)
                half(m + 2, 0)
                return 0

            lax.fori_loop(0, (SUPERS_PER_TILE - 3) // 3, triple, 0)

            half(SUPERS_PER_TILE - 2, (SUPERS_PER_TILE - 2) % 3, with_next2=False)
            half(SUPERS_PER_TILE - 1, (SUPERS_PER_TILE - 1) % 3,
                 with_next=False, with_next2=False)
            drain_s((SUPERS_PER_TILE - 1) % 3)

        @pl.when(c == 0)
        def _():
            edge_loop(lo_hbm)

        @pl.when(c == 1)
        def _():
            edge_loop(hi_hbm)

        plsc.subcore_barrier()
        pltpu.sync_copy(acc.at[pl.ds(rbase, ROWS_PER_TILE)],
                        out_hbm.at[c, pl.ds(rbase, ROWS_PER_TILE)])

    return spmm


_spmm = _make_spmm()


NP8 = ACC_ROWS // 8
_DENSE_BLK = 200
_DENSE_GRID = NP8 // _DENSE_BLK


def _dense_body(s2_ref, elo_ref, ehi_ref, wgl_ref, wgh_ref, wbl_ref, wbh_ref,
                bg_ref, bb_ref, kn_ref, sel_lo_ref, sel_hi_ref, lo_ref, hi_ref):
    sl = s2_ref[0]
    sh = s2_ref[1]
    el = elo_ref[...]
    eh = ehi_ref[...]

    def mm(a, b):
        return jnp.dot(a, b, preferred_element_type=jnp.float32)

    x = (mm(sl, wgl_ref[...]) + mm(sh, wgh_ref[...]) + bg_ref[...]
         + mm(el * sl, wbl_ref[...]) + mm(eh * sh, wbh_ref[...]) + bb_ref[...])
    x = jnp.where(x >= 0, x, 0.2 * x)
    ss = mm(x * x, kn_ref[...])
    x = x / jnp.maximum(jnp.sqrt(ss), 1e-12)
    lo_ref[...] = mm(x, sel_lo_ref[...])
    hi_ref[...] = mm(x, sel_hi_ref[...])


def _dense(s2p, elo, ehi, wgl, wgh, wbl, wbh, bgt, bbt, kn, sel_lo, sel_hi):
    full = lambda shape: pl.BlockSpec(shape, lambda i: tuple(0 for _ in shape))
    return pl.pallas_call(
        _dense_body,
        grid=(_DENSE_GRID,),
        in_specs=[
            pl.BlockSpec((2, _DENSE_BLK, 128), lambda i: (0, i, 0)),
            pl.BlockSpec((_DENSE_BLK, 128), lambda i: (i, 0)),
            pl.BlockSpec((_DENSE_BLK, 128), lambda i: (i, 0)),
            full((128, 256)), full((128, 256)), full((128, 256)), full((128, 256)),
            full((1, 256)), full((1, 256)),
            full((256, 256)), full((256, 128)), full((256, 128)),
        ],
        out_specs=[
            pl.BlockSpec((_DENSE_BLK, 128), lambda i: (i, 0)),
            pl.BlockSpec((_DENSE_BLK, 128), lambda i: (i, 0)),
        ],
        out_shape=[
            jax.ShapeDtypeStruct((NP8, 128), jnp.float32),
            jax.ShapeDtypeStruct((NP8, 128), jnp.float32),
        ],
    )(s2p, elo, ehi, wgl, wgh, wbl, wbh, bgt, bbt, kn, sel_lo, sel_hi)


ROWS_PER_WORKER = BATCH_C // 32
OUT_DIM = DIM * (N_LAYERS_C + 1)


def _make_take3():
    mesh = plsc.VectorSubcoreMesh(core_axis_name="c", subcore_axis_name="s")

    @functools.partial(
        pl.kernel,
        mesh=mesh,
        compiler_params=pltpu.CompilerParams(
            needs_layout_passes=False, use_tc_tiling_on_sc=False),
        out_type=[
            jax.ShapeDtypeStruct((BATCH_C, OUT_DIM), jnp.float32),
            jax.ShapeDtypeStruct((BATCH_C, OUT_DIM), jnp.float32),
            jax.ShapeDtypeStruct((BATCH_C, OUT_DIM), jnp.float32),
        ],
        scratch_types=[
            pltpu.VMEM((ROWS_PER_WORKER,), jnp.int32),
            pltpu.VMEM((8, ROWS_PER_WORKER, HALF), jnp.float32),
            pltpu.SemaphoreType.DMA,
        ],
    )
    def take3(t0, t1, t2, t3, t4, t5, t6, t7, iu_hbm, ii_hbm, ij_hbm,
              ou_hbm, oi_hbm, oj_hbm, idx_v, rbuf, sem):
        tabs = [t0, t1, t2, t3, t4, t5, t6, t7]
        c = lax.axis_index("c")
        s = lax.axis_index("s")
        wid = s * 2 + c
        base = wid * ROWS_PER_WORKER
        for idx_hbm, out_hbm in ((iu_hbm, ou_hbm), (ii_hbm, oi_hbm), (ij_hbm, oj_hbm)):
            pltpu.sync_copy(idx_hbm.at[pl.ds(base, ROWS_PER_WORKER)], idx_v)
            for t in range(8):
                pltpu.async_copy(tabs[t].at[idx_v], rbuf.at[t], sem)
            for t in range(8):
                pltpu.make_async_copy(tabs[t].at[idx_v], rbuf.at[t], sem).wait()
            for t in range(8):
                pltpu.sync_copy(
                    rbuf.at[t],
                    out_hbm.at[pl.ds(base, ROWS_PER_WORKER), pl.ds(t * HALF, HALF)])

    return take3


_take3 = _make_take3()


def kernel(u, i, j, user_embedding, item_embedding, adj_row, adj_col, adj_val,
           W_gc_0, b_gc_0, W_bi_0, b_bi_0, W_gc_1, b_gc_1, W_bi_1, b_bi_1,
           W_gc_2, b_gc_2, W_bi_2, b_bi_2):
    weights = [(W_gc_0, b_gc_0, W_bi_0, b_bi_0),
               (W_gc_1, b_gc_1, W_bi_1, b_bi_1),
               (W_gc_2, b_gc_2, W_bi_2, b_bi_2)]

    pad = NNZ_PAD - NNZ_C
    colp = jnp.concatenate([adj_col.astype(jnp.int32), jnp.zeros((pad,), jnp.int32)])
    rowp = jnp.concatenate([adj_row.astype(jnp.int32), jnp.zeros((pad,), jnp.int32)])
    crp = jnp.stack([colp.reshape(-1, SUPER), rowp.reshape(-1, SUPER)], axis=1)
    valp = jnp.concatenate([adj_val, jnp.zeros((pad,), jnp.float32)])
    zero_rows = jnp.zeros((ROWS_PER_TILE, HALF), jnp.float32)

    zpad = jnp.zeros((ACC_ROWS - N_TOTAL, HALF), jnp.float32)
    lo = jnp.concatenate([user_embedding[:, :HALF], item_embedding[:, :HALF],
                          zpad], axis=0).reshape(NP8, 128)
    hi = jnp.concatenate([user_embedding[:, HALF:], item_embedding[:, HALF:],
                          zpad], axis=0).reshape(NP8, 128)

    eye8 = jnp.eye(8, dtype=jnp.float32)
    m_lo = jnp.concatenate([jnp.eye(HALF, dtype=jnp.float32),
                            jnp.zeros((HALF, HALF), jnp.float32)], axis=0)
    m_hi = jnp.concatenate([jnp.zeros((HALF, HALF), jnp.float32),
                            jnp.eye(HALF, dtype=jnp.float32)], axis=0)
    sel_lo = jnp.kron(eye8, m_lo)
    sel_hi = jnp.kron(eye8, m_hi)
    kn = jnp.kron(eye8, jnp.ones((DIM, DIM), jnp.float32))

    halves = [lo, hi]
    for k in range(N_LAYERS_C):
        wg, bg, wb, bb = weights[k]
        side2 = _spmm(lo.reshape(ACC_ROWS, HALF), hi.reshape(ACC_ROWS, HALF),
                      crp, valp, zero_rows)
        s2p = side2.reshape(2, NP8, 128)
        wgl = jnp.kron(eye8, wg[:HALF, :])
        wgh = jnp.kron(eye8, wg[HALF:, :])
        wbl = jnp.kron(eye8, wb[:HALF, :])
        wbh = jnp.kron(eye8, wb[HALF:, :])
        bgt = jnp.tile(bg, (1, 8))
        bbt = jnp.tile(bb, (1, 8))
        lo, hi = _dense(s2p, lo, hi, wgl, wgh, wbl, wbh, bgt, bbt, kn, sel_lo, sel_hi)
        halves.extend([lo, hi])

    tabs = [h.reshape(ACC_ROWS, HALF) for h in halves]
    iu = u.astype(jnp.int32)
    ii = i.astype(jnp.int32) + N_USERS_C
    ij = j.astype(jnp.int32) + N_USERS_C
    ou, oi, oj = _take3(*tabs, iu, ii, ij)
    return ou, oi, oj


# speedup vs baseline: 1.0358x; 1.0358x over previous
import functools

import jax
import jax.numpy as jnp
from jax import lax
from jax.experimental import pallas as pl
from jax.experimental.pallas import tpu as pltpu
from jax.experimental.pallas import tpu_sc as plsc

N_USERS_C = 50000
N_ITEMS_C = 50000
N_TOTAL = N_USERS_C + N_ITEMS_C
DIM = 32
HALF = 16
NNZ_C = 1600000
BATCH_C = 4096
N_LAYERS_C = 3

NUM_TILES = 16
CHUNK = 128
SUBCH = 4
SUPER = CHUNK * SUBCH
SUPERS_PER_TILE = 201
EDGES_PER_TILE = SUPER * SUPERS_PER_TILE
NNZ_PAD = EDGES_PER_TILE * NUM_TILES
ROWS_PER_TILE = 6300
ACC_ROWS = ROWS_PER_TILE * NUM_TILES


def _make_spmm():
    mesh = plsc.VectorSubcoreMesh(core_axis_name="c", subcore_axis_name="s")

    @functools.partial(
        pl.kernel,
        mesh=mesh,
        compiler_params=pltpu.CompilerParams(
            needs_layout_passes=False, use_tc_tiling_on_sc=False),
        out_type=jax.ShapeDtypeStruct((2, ACC_ROWS, HALF), jnp.float32),
        scratch_types=(
            [pltpu.VMEM((SUPER,), jnp.int32) for _ in range(3)]
            + [pltpu.VMEM((SUPER,), jnp.int32) for _ in range(3)]
            + [pltpu.VMEM((SUPER,), jnp.float32) for _ in range(3)]
            + [pltpu.VMEM((SUPER, HALF), jnp.float32) for _ in range(3)]
            + [pltpu.VMEM_SHARED((ACC_ROWS, HALF), jnp.float32)]
            + [pltpu.SemaphoreType.DMA for _ in range(9)]
        ),
    )
    def spmm(lo_hbm, hi_hbm, col_hbm, row_hbm, val_hbm, zero_hbm, out_hbm,
             c0, c1, c2, r0, r1, r2, v0, v1, v2, g0, g1, g2, acc,
             ls0, ls1, ls2, gs0, gs1, gs2, ss0, ss1, ss2):
        col_v = [c0, c1, c2]
        row_v = [r0, r1, r2]
        val_v = [v0, v1, v2]
        rows_v = [g0, g1, g2]
        lsem = [ls0, ls1, ls2]
        gsem = [gs0, gs1, gs2]
        ssem = [ss0, ss1, ss2]
        c = lax.axis_index("c")
        s = lax.axis_index("s")
        rbase = s * ROWS_PER_TILE
        pltpu.sync_copy(zero_hbm, acc.at[pl.ds(rbase, ROWS_PER_TILE)])
        plsc.subcore_barrier()

        sbase = s * SUPERS_PER_TILE
        esplats = [jnp.full((16,), e, jnp.int32) for e in range(16)]

        def edge_loop(table):
            def fire_l(t, m):
                off = (sbase + m) * SUPER
                pltpu.async_copy(col_hbm.at[pl.ds(off, SUPER)], col_v[t], lsem[t])
                pltpu.async_copy(row_hbm.at[pl.ds(off, SUPER)], row_v[t], lsem[t])
                pltpu.async_copy(val_hbm.at[pl.ds(off, SUPER)], val_v[t], lsem[t])

            def drain_l(t, m):
                off = (sbase + m) * SUPER
                pltpu.make_async_copy(col_hbm.at[pl.ds(off, SUPER)], col_v[t], lsem[t]).wait()
                pltpu.make_async_copy(row_hbm.at[pl.ds(off, SUPER)], row_v[t], lsem[t]).wait()
                pltpu.make_async_copy(val_hbm.at[pl.ds(off, SUPER)], val_v[t], lsem[t]).wait()

            def fire_g(t):
                pltpu.async_copy(table.at[col_v[t]], rows_v[t], gsem[t])

            def drain_g(t):
                pltpu.make_async_copy(table.at[col_v[t]], rows_v[t], gsem[t]).wait()

            def fire_s(t):
                pltpu.async_copy(rows_v[t], acc.at[row_v[t]], ssem[t], add=True)

            def drain_s(t):
                pltpu.make_async_copy(rows_v[t], acc.at[row_v[t]], ssem[t]).wait()

            def scale(t):
                def body(g, _):
                    vv16 = val_v[t][pl.ds(g * 16, 16)]
                    for e in range(16):
                        r = g * 16 + e
                        vv = jnp.take(vv16, esplats[e])
                        rows_v[t][r, :] = rows_v[t][r, :] * vv
                    return 0

                lax.fori_loop(0, SUPER // 16, body, 0)

            def half(m, t, with_drain_s=True, with_next=True, with_next2=True):
                t1 = (t + 1) % 3
                t2 = (t + 2) % 3
                if with_next:
                    drain_l(t1, m + 1)
                    fire_g(t1)
                drain_g(t)
                scale(t)
                fire_s(t)
                if with_drain_s:
                    drain_s(t2)
                if with_next2:
                    fire_l(t2, m + 2)

            fire_l(0, 0)
            drain_l(0, 0)
            fire_g(0)
            fire_l(1, 1)
            half(0, 0, with_drain_s=False)

            def triple(k, _):
                m = 3 * k + 1
                half(m, 1)
                half(m + 1, 2)
                half(m + 2, 0)
                return 0

            lax.fori_loop(0, (SUPERS_PER_TILE - 3) // 3, triple, 0)

            half(SUPERS_PER_TILE - 2, (SUPERS_PER_TILE - 2) % 3, with_next2=False)
            half(SUPERS_PER_TILE - 1, (SUPERS_PER_TILE - 1) % 3,
                 with_next=False, with_next2=False)
            drain_s((SUPERS_PER_TILE - 1) % 3)

        @pl.when(c == 0)
        def _():
            edge_loop(lo_hbm)

        @pl.when(c == 1)
        def _():
            edge_loop(hi_hbm)

        plsc.subcore_barrier()
        pltpu.sync_copy(acc.at[pl.ds(rbase, ROWS_PER_TILE)],
                        out_hbm.at[c, pl.ds(rbase, ROWS_PER_TILE)])

    return spmm


_spmm = _make_spmm()


NP8 = ACC_ROWS // 8
_DENSE_BLK = 200
_DENSE_GRID = NP8 // _DENSE_BLK


def _dense_body(s2_ref, elo_ref, ehi_ref, wgl_ref, wgh_ref, wbl_ref, wbh_ref,
                bg_ref, bb_ref, kn_ref, sel_lo_ref, sel_hi_ref, lo_ref, hi_ref):
    sl = s2_ref[0]
    sh = s2_ref[1]
    el = elo_ref[...]
    eh = ehi_ref[...]

    def mm(a, b):
        return jnp.dot(a, b, preferred_element_type=jnp.float32)

    x = (mm(sl, wgl_ref[...]) + mm(sh, wgh_ref[...]) + bg_ref[...]
         + mm(el * sl, wbl_ref[...]) + mm(eh * sh, wbh_ref[...]) + bb_ref[...])
    x = jnp.where(x >= 0, x, 0.2 * x)
    ss = mm(x * x, kn_ref[...])
    x = x / jnp.maximum(jnp.sqrt(ss), 1e-12)
    lo_ref[...] = mm(x, sel_lo_ref[...])
    hi_ref[...] = mm(x, sel_hi_ref[...])


def _dense(s2p, elo, ehi, wgl, wgh, wbl, wbh, bgt, bbt, kn, sel_lo, sel_hi):
    full = lambda shape: pl.BlockSpec(shape, lambda i: tuple(0 for _ in shape))
    return pl.pallas_call(
        _dense_body,
        grid=(_DENSE_GRID,),
        in_specs=[
            pl.BlockSpec((2, _DENSE_BLK, 128), lambda i: (0, i, 0)),
            pl.BlockSpec((_DENSE_BLK, 128), lambda i: (i, 0)),
            pl.BlockSpec((_DENSE_BLK, 128), lambda i: (i, 0)),
            full((128, 256)), full((128, 256)), full((128, 256)), full((128, 256)),
            full((1, 256)), full((1, 256)),
            full((256, 256)), full((256, 128)), full((256, 128)),
        ],
        out_specs=[
            pl.BlockSpec((_DENSE_BLK, 128), lambda i: (i, 0)),
            pl.BlockSpec((_DENSE_BLK, 128), lambda i: (i, 0)),
        ],
        out_shape=[
            jax.ShapeDtypeStruct((NP8, 128), jnp.float32),
            jax.ShapeDtypeStruct((NP8, 128), jnp.float32),
        ],
    )(s2p, elo, ehi, wgl, wgh, wbl, wbh, bgt, bbt, kn, sel_lo, sel_hi)


ROWS_PER_WORKER = BATCH_C // 32
OUT_DIM = DIM * (N_LAYERS_C + 1)


def _make_take3():
    mesh = plsc.VectorSubcoreMesh(core_axis_name="c", subcore_axis_name="s")

    @functools.partial(
        pl.kernel,
        mesh=mesh,
        compiler_params=pltpu.CompilerParams(
            needs_layout_passes=False, use_tc_tiling_on_sc=False),
        out_type=[
            jax.ShapeDtypeStruct((BATCH_C, OUT_DIM), jnp.float32),
            jax.ShapeDtypeStruct((BATCH_C, OUT_DIM), jnp.float32),
            jax.ShapeDtypeStruct((BATCH_C, OUT_DIM), jnp.float32),
        ],
        scratch_types=[
            pltpu.VMEM((ROWS_PER_WORKER,), jnp.int32),
            pltpu.VMEM((8, ROWS_PER_WORKER, HALF), jnp.float32),
            pltpu.SemaphoreType.DMA,
        ],
    )
    def take3(t0, t1, t2, t3, t4, t5, t6, t7, iu_hbm, ii_hbm, ij_hbm,
              ou_hbm, oi_hbm, oj_hbm, idx_v, rbuf, sem):
        tabs = [t0, t1, t2, t3, t4, t5, t6, t7]
        c = lax.axis_index("c")
        s = lax.axis_index("s")
        wid = s * 2 + c
        base = wid * ROWS_PER_WORKER
        for idx_hbm, out_hbm in ((iu_hbm, ou_hbm), (ii_hbm, oi_hbm), (ij_hbm, oj_hbm)):
            pltpu.sync_copy(idx_hbm.at[pl.ds(base, ROWS_PER_WORKER)], idx_v)
            for t in range(8):
                pltpu.async_copy(tabs[t].at[idx_v], rbuf.at[t], sem)
            for t in range(8):
                pltpu.make_async_copy(tabs[t].at[idx_v], rbuf.at[t], sem).wait()
            for t in range(8):
                pltpu.sync_copy(
                    rbuf.at[t],
                    out_hbm.at[pl.ds(base, ROWS_PER_WORKER), pl.ds(t * HALF, HALF)])

    return take3


_take3 = _make_take3()


def kernel(u, i, j, user_embedding, item_embedding, adj_row, adj_col, adj_val,
           W_gc_0, b_gc_0, W_bi_0, b_bi_0, W_gc_1, b_gc_1, W_bi_1, b_bi_1,
           W_gc_2, b_gc_2, W_bi_2, b_bi_2):
    weights = [(W_gc_0, b_gc_0, W_bi_0, b_bi_0),
               (W_gc_1, b_gc_1, W_bi_1, b_bi_1),
               (W_gc_2, b_gc_2, W_bi_2, b_bi_2)]

    pad = NNZ_PAD - NNZ_C
    colp = jnp.concatenate([adj_col.astype(jnp.int32), jnp.zeros((pad,), jnp.int32)])
    rowp = jnp.concatenate([adj_row.astype(jnp.int32), jnp.zeros((pad,), jnp.int32)])
    valp = jnp.concatenate([adj_val, jnp.zeros((pad,), jnp.float32)])
    zero_rows = jnp.zeros((ROWS_PER_TILE, HALF), jnp.float32)

    zpad = jnp.zeros((ACC_ROWS - N_TOTAL, HALF), jnp.float32)
    lo = jnp.concatenate([user_embedding[:, :HALF], item_embedding[:, :HALF],
                          zpad], axis=0).reshape(NP8, 128)
    hi = jnp.concatenate([user_embedding[:, HALF:], item_embedding[:, HALF:],
                          zpad], axis=0).reshape(NP8, 128)

    eye8 = jnp.eye(8, dtype=jnp.float32)
    m_lo = jnp.concatenate([jnp.eye(HALF, dtype=jnp.float32),
                            jnp.zeros((HALF, HALF), jnp.float32)], axis=0)
    m_hi = jnp.concatenate([jnp.zeros((HALF, HALF), jnp.float32),
                            jnp.eye(HALF, dtype=jnp.float32)], axis=0)
    sel_lo = jnp.kron(eye8, m_lo)
    sel_hi = jnp.kron(eye8, m_hi)
    kn = jnp.kron(eye8, jnp.ones((DIM, DIM), jnp.float32))

    halves = [lo, hi]
    for k in range(N_LAYERS_C):
        wg, bg, wb, bb = weights[k]
        side2 = _spmm(lo.reshape(ACC_ROWS, HALF), hi.reshape(ACC_ROWS, HALF),
                      colp, rowp, valp, zero_rows)
        s2p = side2.reshape(2, NP8, 128)
        wgl = jnp.kron(eye8, wg[:HALF, :])
        wgh = jnp.kron(eye8, wg[HALF:, :])
        wbl = jnp.kron(eye8, wb[:HALF, :])
        wbh = jnp.kron(eye8, wb[HALF:, :])
        bgt = jnp.tile(bg, (1, 8))
        bbt = jnp.tile(bb, (1, 8))
        lo, hi = _dense(s2p, lo, hi, wgl, wgh, wbl, wbh, bgt, bbt, kn, sel_lo, sel_hi)
        halves.extend([lo, hi])

    tabs = [h.reshape(ACC_ROWS, HALF) for h in halves]
    iu = u.astype(jnp.int32)
    ii = i.astype(jnp.int32) + N_USERS_C
    ij = j.astype(jnp.int32) + N_USERS_C
    ou, oi, oj = _take3(*tabs, iu, ii, ij)
    return ou, oi, oj
